# baseline (speedup 1.0000x reference)
import functools

import jax
import jax.numpy as jnp
from jax import lax
from jax.experimental import pallas as pl
from jax.experimental.pallas import tpu as pltpu
from jax.experimental.pallas import tpu_sc as plsc

N = 10000
E = 320000
D = 128
DH = D // 2

NC = 2
NS = 16
NW = NC * NS

CHUNK = 128
BLK = 20
NBLK = 8
NCH = BLK * NBLK
EPT = NCH * CHUNK
REAL = E // NS
DUMP = N
N_PAD = 10240
TILE_N = N_PAD // NS
LAST = N - (NS - 1) * TILE_N
RV = 2 * CHUNK
DEG_CH = NCH // NC

RB = 2048
GRID = (N + RB - 1) // RB
_f32 = jnp.float32

_mesh = plsc.VectorSubcoreMesh(core_axis_name="c", subcore_axis_name="s")


@functools.partial(
    pl.kernel,
    out_type=jax.ShapeDtypeStruct((NC, N_PAD), _f32),
    mesh=_mesh,
    scratch_types=[
        pltpu.VMEM((DEG_CH, 2, CHUNK), jnp.int32),
        pltpu.VMEM((CHUNK,), _f32),
        pltpu.VMEM((TILE_N,), _f32),
        pltpu.VMEM_SHARED((N_PAD,), _f32),
    ],
)
def _deg_kernel(idx_hbm, out_hbm, idx_v, ones_v, zer_v, deg_sh):
    ci = lax.axis_index("c")
    si = lax.axis_index("s")

    @pl.loop(0, CHUNK, step=16)
    def _(i):
        ones_v[pl.ds(i, 16)] = jnp.ones((16,), _f32)

    @pl.loop(0, TILE_N, step=16)
    def _(i):
        zer_v[pl.ds(i, 16)] = jnp.zeros((16,), _f32)

    pltpu.sync_copy(zer_v, deg_sh.at[pl.ds(si * TILE_N, TILE_N)])
    pltpu.sync_copy(idx_hbm.at[si, pl.ds(ci * DEG_CH, DEG_CH)], idx_v)
    plsc.subcore_barrier()

    @pl.loop(0, DEG_CH)
    def _(ch):
        pltpu.sync_copy(ones_v, deg_sh.at[idx_v.at[ch, 1]], add=True)

    plsc.subcore_barrier()
    pltpu.sync_copy(deg_sh.at[pl.ds(si * TILE_N, TILE_N)],
                    out_hbm.at[ci, pl.ds(si * TILE_N, TILE_N)])


@functools.partial(
    pl.kernel,
    out_type=jax.ShapeDtypeStruct((NC, N_PAD, DH), _f32),
    mesh=_mesh,
    scratch_types=[
        pltpu.VMEM((2, BLK, 2, CHUNK), jnp.int32),
        pltpu.VMEM((RV, DH), _f32),
        pltpu.VMEM_SHARED((N_PAD, DH), _f32),
        pltpu.SemaphoreType.DMA,
        pltpu.SemaphoreType.DMA,
        pltpu.SemaphoreType.DMA,
        pltpu.SemaphoreType.DMA,
        pltpu.SemaphoreType.DMA,
        pltpu.SemaphoreType.DMA,
    ],
)
def _mp_kernel(hs_hbm, idx_hbm, out_hbm,
               ring_v, rows_v, hs_sh, g0, g1, s0, s1, i0, i1):
    g_sems = (g0, g1)
    s_sems = (s0, s1)
    i_sems = (i0, i1)
    ci = lax.axis_index("c")
    si = lax.axis_index("s")
    bufs = (rows_v.at[pl.ds(0, CHUNK)], rows_v.at[pl.ds(CHUNK, CHUNK)])
    base = si * TILE_N

    @pl.when(si < NS - 1)
    def _():
        pltpu.sync_copy(hs_hbm.at[ci, pl.ds(base, TILE_N)],
                        hs_sh.at[pl.ds(base, TILE_N)])

    @pl.when(si == NS - 1)
    def _():
        pltpu.sync_copy(hs_hbm.at[ci, pl.ds((NS - 1) * TILE_N, LAST)],
                        hs_sh.at[pl.ds((NS - 1) * TILE_N, LAST)])

    @pl.loop(0, RV)
    def _(r):
        @pl.loop(0, DH, step=16)
        def _(cc):
            rows_v[r, pl.ds(cc, 16)] = jnp.zeros((16,), _f32)


    pltpu.sync_copy(idx_hbm.at[si, pl.ds(0, BLK)], ring_v.at[0])
    plsc.subcore_barrier()
    for b in range(2):
        pltpu.async_copy(hs_sh.at[ring_v.at[0, b, 0]], bufs[b], g_sems[b])

    def wait_gather(idx_row, b):
        pltpu.make_async_copy(hs_sh.at[idx_row], bufs[b], g_sems[b]).wait()

    for j in range(NBLK):
        rj = ring_v.at[j % 2]
        rn = ring_v.at[(j + 1) % 2]
        if j < NBLK - 1:
            pltpu.async_copy(idx_hbm.at[si, pl.ds((j + 1) * BLK, BLK)],
                             rn, i_sems[(j + 1) % 2])

        @pl.loop(0, BLK - 2, step=2)
        def _(c):
            for b in range(2):
                wait_gather(rj.at[c + b, 0], b)
                pltpu.async_copy(hs_sh.at[rj.at[c + 2 + b, 0]],
                                 bufs[b], g_sems[b])

        for b in range(2):
            wait_gather(rj.at[BLK - 2 + b, 0], b)
        if j < NBLK - 1:
            pltpu.make_async_copy(idx_hbm.at[si, pl.ds((j + 1) * BLK, BLK)],
                                  rn, i_sems[(j + 1) % 2]).wait()
            for b in range(2):
                pltpu.async_copy(hs_sh.at[rn.at[b, 0]], bufs[b], g_sems[b])

    plsc.subcore_barrier()
    pltpu.sync_copy(hs_sh.at[pl.ds(base, TILE_N)],
                    out_hbm.at[ci, pl.ds(base, TILE_N)])


def _mmT(x, w):
    return lax.dot_general(x, w, (((1,), (1,)), ((), ())),
                           preferred_element_type=_f32)


def _dinv_of(deg_r):
    return lax.rsqrt(deg_r[0, :] + deg_r[1, :] + 1.0)


def _halves(r):
    return jnp.concatenate([r[0], r[1]], axis=1)


def _dense1_body(x_r, w0_r, ws_r, bs_r, deg_r, hs0_r, skip_r):
    dinv = _dinv_of(deg_r)
    x = x_r[...]
    hs = _mmT(x, w0_r[...]) * dinv[:, None]
    hs0_r[0] = hs[:, :DH]
    hs0_r[1] = hs[:, DH:]
    skip_r[...] = _mmT(x, ws_r[...]) + bs_r[...]


def _dense2_body(p_r, hs0_r, skip_r, deg_r, b0_r, a_r, w1_r, hs1_r):
    dinv = _dinv_of(deg_r)
    agg = (_halves(p_r) + _halves(hs0_r)) * dinv[:, None] + b0_r[...]
    h1 = jnp.where(agg >= 0, agg, a_r[...] * agg)
    u = skip_r[...] + h1
    hs1 = _mmT(u, w1_r[...]) * dinv[:, None]
    hs1_r[0] = hs1[:, :DH]
    hs1_r[1] = hs1[:, DH:]


def _dense3_body(q_r, hs1_r, deg_r, b1_r, a_r, out_r):
    dinv = _dinv_of(deg_r)
    agg = (_halves(q_r) + _halves(hs1_r)) * dinv[:, None] + b1_r[...]
    out_r[...] = jnp.where(agg >= 0, agg, a_r[...] * agg)


_row = lambda: pl.BlockSpec((RB, D), lambda i: (i, 0))
_full = lambda: pl.BlockSpec((D, D), lambda i: (0, 0))
_vec = lambda: pl.BlockSpec((1, D), lambda i: (0, 0))
_degb = lambda: pl.BlockSpec((NC, RB), lambda i: (0, i))
_half = lambda: pl.BlockSpec((NC, RB, DH), lambda i: (0, i, 0))
_nd = lambda: jax.ShapeDtypeStruct((N, D), _f32)
_nh = lambda: jax.ShapeDtypeStruct((NC, N, DH), _f32)


def _dense1(x, W0, Ws, bs2, degp):
    return pl.pallas_call(
        _dense1_body,
        grid=(GRID,),
        in_specs=[_row(), _full(), _full(), _vec(), _degb()],
        out_specs=[_half(), _row()],
        out_shape=[_nh(), _nd()],
    )(x, W0, Ws, bs2, degp)


def _dense2(p, hs0, skip, degp, b02, a2, W1):
    return pl.pallas_call(
        _dense2_body,
        grid=(GRID,),
        in_specs=[_half(), _half(), _row(), _degb(), _vec(), _vec(), _full()],
        out_specs=_half(),
        out_shape=_nh(),
    )(p, hs0, skip, degp, b02, a2, W1)


def _dense3(q, hs1, degp, b12, a2):
    return pl.pallas_call(
        _dense3_body,
        grid=(GRID,),
        in_specs=[_half(), _half(), _degb(), _vec(), _vec()],
        out_specs=_row(),
        out_shape=_nd(),
    )(q, hs1, degp, b12, a2)


def kernel(x, edge_index, W0, b0, W1, b1, Ws, bs, a):
    src = edge_index[0].astype(jnp.int32)
    dst = edge_index[1].astype(jnp.int32)
    padt = EPT - REAL
    spad = jnp.zeros((NS, padt), jnp.int32)
    dpad = DUMP + (jnp.arange(padt, dtype=jnp.int32)[None, :]
                   + 13 * jnp.arange(NS, dtype=jnp.int32)[:, None]) % (N_PAD - N)
    srcp = jnp.concatenate([src.reshape(NS, REAL), spad], axis=1)
    dstp = jnp.concatenate([dst.reshape(NS, REAL), dpad], axis=1)
    idxp = jnp.stack([srcp.reshape(NS, NCH, CHUNK),
                      dstp.reshape(NS, NCH, CHUNK)], axis=2)

    bs2 = bs.reshape(1, D)
    b02 = b0.reshape(1, D)
    b12 = b1.reshape(1, D)
    a2 = a.reshape(1, D)

    degp = _deg_kernel(idxp)
    hs0, skip = _dense1(x, W0, Ws, bs2, degp)
    p = _mp_kernel(hs0, idxp)
    hs1 = _dense2(p, hs0, skip, degp, b02, a2, W1)
    q = _mp_kernel(hs1, idxp)
    return _dense3(q, hs1, degp, b12, a2)
